# baseline (speedup 1.0000x reference)
import functools

import jax
import jax.numpy as jnp
from jax import lax
from jax.experimental import pallas as pl
from jax.experimental.pallas import tpu as pltpu
from jax.experimental.pallas import tpu_sc as plsc

HIDDEN = 64
VOCAB = 2016
SCALER = 7

NC = 2
NS = 16
NW = NC * NS
L = 16

B, T = 4096, 200
TOTAL = B * T
PER_W = TOTAL // NW
ROWS_W = B // NW
PIECE = 3200
NPIECE = PER_W // PIECE
SBR = 2
NSB = ROWS_W // SBR
G0, G1 = 128, 72


def _table_body(emb_ref, w_ref, b_ref, out_ref):
    t = lax.dot_general(
        emb_ref[:], w_ref[:], (((1,), (1,)), ((), ())),
        preferred_element_type=jnp.float32,
    )
    t = t + b_ref[:]
    out_ref[:] = jnp.concatenate([t[:, 1:], jnp.sin(t[:, :1])], axis=1)


def _build_table(emb, W, b):
    return pl.pallas_call(
        _table_body,
        out_shape=jax.ShapeDtypeStruct((VOCAB, HIDDEN), jnp.float32),
    )(emb, W, b.reshape(1, HIDDEN))


@functools.partial(
    pl.kernel,
    mesh=plsc.VectorSubcoreMesh(core_axis_name="c", subcore_axis_name="s"),
    compiler_params=pltpu.CompilerParams(use_tc_tiling_on_sc=False),
    out_type=jax.ShapeDtypeStruct((B, T, HIDDEN), jnp.float32),
    scratch_types=[
        pltpu.VMEM((PIECE,), jnp.int32),
        pltpu.VMEM((PIECE,), jnp.int32),
        pltpu.VMEM((PER_W,), jnp.int32),
        pltpu.VMEM((2, SBR, T, HIDDEN), jnp.float32),
        pltpu.VMEM_SHARED((VOCAB, HIDDEN), jnp.float32),
        pltpu.SemaphoreType.DMA,
        pltpu.SemaphoreType.DMA,
        pltpu.SemaphoreType.DMA,
        pltpu.SemaphoreType.DMA,
        pltpu.SemaphoreType.DMA,
    ],
)
def _sc_gather(tod_hbm, dow_hbm, table_hbm, out_hbm,
               tc_v, dc_v, idx_v, rows_v, table_sh,
               ssem, gsem0, gsem1, wsem0, wsem1):
    sid = lax.axis_index("s")
    wid = sid * NC + lax.axis_index("c")
    base = wid * PER_W
    row0 = wid * ROWS_W
    gsem = (gsem0, gsem1)
    wsem = (wsem0, wsem1)

    @pl.when(sid == 0)
    def _():
        pltpu.sync_copy(table_hbm, table_sh)

    def piece(p, carry):
        poff = p * PIECE
        ct = pltpu.async_copy(tod_hbm.at[pl.ds(base + poff, PIECE)], tc_v,
                              ssem)
        cd = pltpu.async_copy(dow_hbm.at[pl.ds(base + poff, PIECE)], dc_v,
                              ssem)
        ct.wait()
        cd.wait()

        def vec(i, c):
            s = pl.ds(i * L, L)
            idx_v[pl.ds(poff + i * L, L)] = tc_v[s] * SCALER + dc_v[s]
            return c

        lax.fori_loop(0, PIECE // L, vec, 0)
        return carry

    lax.fori_loop(0, NPIECE, piece, 0)
    plsc.subcore_barrier()

    def start_gathers(sb, b):
        for r in range(SBR):
            toff = sb * SBR * T + r * T
            pltpu.async_copy(
                table_sh.at[idx_v.at[pl.ds(toff, G0)]],
                rows_v.at[b, r, pl.ds(0, G0)], gsem[b])
            pltpu.async_copy(
                table_sh.at[idx_v.at[pl.ds(toff + G0, G1)]],
                rows_v.at[b, r, pl.ds(G0, G1)], gsem[b])

    def wait_gathers(b):
        for r in range(SBR):
            pltpu.make_async_copy(
                table_sh.at[idx_v.at[pl.ds(0, G0)]],
                rows_v.at[b, r, pl.ds(0, G0)], gsem[b]).wait()
            pltpu.make_async_copy(
                table_sh.at[idx_v.at[pl.ds(0, G1)]],
                rows_v.at[b, r, pl.ds(G0, G1)], gsem[b]).wait()

    def start_write(sb, b):
        pltpu.async_copy(rows_v.at[b],
                         out_hbm.at[pl.ds(row0 + sb * SBR, SBR)], wsem[b])

    def wait_write(b):
        pltpu.make_async_copy(rows_v.at[b],
                              out_hbm.at[pl.ds(row0, SBR)], wsem[b]).wait()

    start_gathers(0, 0)

    def body(p, carry):
        s0 = p * 2
        s1 = s0 + 1
        wait_gathers(0)
        start_write(s0, 0)

        @pl.when(p > 0)
        def _():
            wait_write(1)
        start_gathers(s1, 1)
        wait_gathers(1)
        start_write(s1, 1)

        @pl.when(p + 1 < NSB // 2)
        def _():
            wait_write(0)
            start_gathers(s0 + 2, 0)
        return carry

    lax.fori_loop(0, NSB // 2, body, 0)
    wait_write(0)
    wait_write(1)


def kernel(tod, dow, emb, W, b):
    table = _build_table(emb, W, b)
    return _sc_gather(tod.reshape(-1), dow.reshape(-1), table)


# speedup vs baseline: 1.0016x; 1.0016x over previous
import functools

import jax
import jax.numpy as jnp
from jax import lax
from jax.experimental import pallas as pl
from jax.experimental.pallas import tpu as pltpu
from jax.experimental.pallas import tpu_sc as plsc

HIDDEN = 64
VOCAB = 2016
SCALER = 7

NC = 2
NS = 16
NW = NC * NS
L = 16

B, T = 4096, 200
TOTAL = B * T
PER_W = TOTAL // NW
ROWS_W = B // NW
PIECE = 3200
NPIECE = PER_W // PIECE
G0 = 128
FB_W = PER_W // G0
SBK = 4
NSB = FB_W // SBK


def _table_body(emb_ref, w_ref, b_ref, out_ref):
    t = lax.dot_general(
        emb_ref[:], w_ref[:], (((1,), (1,)), ((), ())),
        preferred_element_type=jnp.float32,
    )
    t = t + b_ref[:]
    out_ref[:] = jnp.concatenate([t[:, 1:], jnp.sin(t[:, :1])], axis=1)


def _build_table(emb, W, b):
    return pl.pallas_call(
        _table_body,
        out_shape=jax.ShapeDtypeStruct((VOCAB, HIDDEN), jnp.float32),
    )(emb, W, b.reshape(1, HIDDEN))


@functools.partial(
    pl.kernel,
    mesh=plsc.VectorSubcoreMesh(core_axis_name="c", subcore_axis_name="s"),
    compiler_params=pltpu.CompilerParams(use_tc_tiling_on_sc=False),
    out_type=jax.ShapeDtypeStruct((TOTAL // G0, G0, HIDDEN), jnp.float32),
    scratch_types=[
        pltpu.VMEM((PIECE,), jnp.int32),
        pltpu.VMEM((PIECE,), jnp.int32),
        pltpu.VMEM((PER_W,), jnp.int32),
        pltpu.VMEM((2, SBK, G0, HIDDEN), jnp.float32),
        pltpu.VMEM_SHARED((VOCAB, HIDDEN), jnp.float32),
        pltpu.SemaphoreType.DMA,
        pltpu.SemaphoreType.DMA,
        pltpu.SemaphoreType.DMA,
        pltpu.SemaphoreType.DMA,
        pltpu.SemaphoreType.DMA,
    ],
)
def _sc_gather(tod_hbm, dow_hbm, table_hbm, out_hbm,
               tc_v, dc_v, idx_v, rows_v, table_sh,
               ssem, gsem0, gsem1, wsem0, wsem1):
    sid = lax.axis_index("s")
    wid = sid * NC + lax.axis_index("c")
    base = wid * PER_W
    fb0 = wid * FB_W
    gsem = (gsem0, gsem1)
    wsem = (wsem0, wsem1)

    @pl.when(sid == 0)
    def _():
        pltpu.sync_copy(table_hbm, table_sh)

    def piece(p, carry):
        poff = p * PIECE
        ct = pltpu.async_copy(tod_hbm.at[pl.ds(base + poff, PIECE)], tc_v,
                              ssem)
        cd = pltpu.async_copy(dow_hbm.at[pl.ds(base + poff, PIECE)], dc_v,
                              ssem)
        ct.wait()
        cd.wait()

        def vec(i, c):
            s = pl.ds(i * L, L)
            idx_v[pl.ds(poff + i * L, L)] = tc_v[s] * SCALER + dc_v[s]
            return c

        lax.fori_loop(0, PIECE // L, vec, 0)
        return carry

    lax.fori_loop(0, NPIECE, piece, 0)
    plsc.subcore_barrier()

    def start_gathers(sb, b):
        for k in range(SBK):
            toff = sb * SBK * G0 + k * G0
            pltpu.async_copy(
                table_sh.at[idx_v.at[pl.ds(toff, G0)]],
                rows_v.at[b, k], gsem[b])

    def wait_gathers(b):
        for k in range(SBK):
            pltpu.make_async_copy(
                table_sh.at[idx_v.at[pl.ds(0, G0)]],
                rows_v.at[b, k], gsem[b]).wait()

    def start_write(sb, b):
        pltpu.async_copy(rows_v.at[b],
                         out_hbm.at[pl.ds(fb0 + sb * SBK, SBK)], wsem[b])

    def wait_write(b):
        pltpu.make_async_copy(rows_v.at[b],
                              out_hbm.at[pl.ds(fb0, SBK)], wsem[b]).wait()

    start_gathers(0, 0)

    def body(p, carry):
        s0 = p * 2
        s1 = s0 + 1
        wait_gathers(0)
        start_write(s0, 0)

        @pl.when(p > 0)
        def _():
            wait_write(1)
        start_gathers(s1, 1)
        wait_gathers(1)
        start_write(s1, 1)

        @pl.when(p + 1 < NSB // 2)
        def _():
            wait_write(0)
            start_gathers(s0 + 2, 0)
        return carry

    lax.fori_loop(0, NSB // 2, body, 0)
    wait_write(0)
    wait_write(1)


def kernel(tod, dow, emb, W, b):
    table = _build_table(emb, W, b)
    out = _sc_gather(tod.reshape(-1), dow.reshape(-1), table)
    return out.reshape(B, T, HIDDEN)


# speedup vs baseline: 1.0325x; 1.0308x over previous
import functools

import jax
import jax.numpy as jnp
from jax import lax
from jax.experimental import pallas as pl
from jax.experimental.pallas import tpu as pltpu
from jax.experimental.pallas import tpu_sc as plsc

HIDDEN = 64
VOCAB = 2016
SCALER = 7

NC = 2
NS = 16
NW = NC * NS

B, T = 4096, 200
TOTAL = B * T
PER_W = TOTAL // NW
G0 = 128
FB_W = PER_W // G0
SBK = 4
NSB = FB_W // SBK


def _prep_body(emb_ref, w_ref, b_ref, tod_ref, dow_ref, table_ref, idx_ref):
    t = lax.dot_general(
        emb_ref[:], w_ref[:], (((1,), (1,)), ((), ())),
        preferred_element_type=jnp.float32,
    )
    t = t + b_ref[:]
    table_ref[:] = jnp.concatenate([t[:, 1:], jnp.sin(t[:, :1])], axis=1)
    idx_ref[:] = tod_ref[:] * SCALER + dow_ref[:]


def _prep(emb, W, b, tod, dow):
    return pl.pallas_call(
        _prep_body,
        out_shape=(
            jax.ShapeDtypeStruct((VOCAB, HIDDEN), jnp.float32),
            jax.ShapeDtypeStruct((B, T), jnp.int32),
        ),
    )(emb, W, b.reshape(1, HIDDEN), tod, dow)


@functools.partial(
    pl.kernel,
    mesh=plsc.VectorSubcoreMesh(core_axis_name="c", subcore_axis_name="s"),
    compiler_params=pltpu.CompilerParams(use_tc_tiling_on_sc=False),
    out_type=jax.ShapeDtypeStruct((TOTAL // G0, G0, HIDDEN), jnp.float32),
    scratch_types=[
        pltpu.VMEM((PER_W,), jnp.int32),
        pltpu.VMEM((2, SBK, G0, HIDDEN), jnp.float32),
        pltpu.VMEM_SHARED((VOCAB, HIDDEN), jnp.float32),
        pltpu.SemaphoreType.DMA,
        pltpu.SemaphoreType.DMA,
        pltpu.SemaphoreType.DMA,
        pltpu.SemaphoreType.DMA,
        pltpu.SemaphoreType.DMA,
    ],
)
def _sc_gather(idx_hbm, table_hbm, out_hbm,
               idx_v, rows_v, table_sh,
               ssem, gsem0, gsem1, wsem0, wsem1):
    sid = lax.axis_index("s")
    wid = sid * NC + lax.axis_index("c")
    base = wid * PER_W
    fb0 = wid * FB_W
    gsem = (gsem0, gsem1)
    wsem = (wsem0, wsem1)

    cp = pltpu.async_copy(idx_hbm.at[pl.ds(base, PER_W)], idx_v, ssem)

    @pl.when(sid == 0)
    def _():
        pltpu.sync_copy(table_hbm, table_sh)

    plsc.subcore_barrier()
    cp.wait()

    def start_gathers(sb, b):
        for k in range(SBK):
            toff = sb * SBK * G0 + k * G0
            pltpu.async_copy(
                table_sh.at[idx_v.at[pl.ds(toff, G0)]],
                rows_v.at[b, k], gsem[b])

    def wait_gathers(b):
        for k in range(SBK):
            pltpu.make_async_copy(
                table_sh.at[idx_v.at[pl.ds(0, G0)]],
                rows_v.at[b, k], gsem[b]).wait()

    def start_write(sb, b):
        pltpu.async_copy(rows_v.at[b],
                         out_hbm.at[pl.ds(fb0 + sb * SBK, SBK)], wsem[b])

    def wait_write(b):
        pltpu.make_async_copy(rows_v.at[b],
                              out_hbm.at[pl.ds(fb0, SBK)], wsem[b]).wait()

    start_gathers(0, 0)

    def body(p, carry):
        s0 = p * 2
        s1 = s0 + 1
        wait_gathers(0)
        start_write(s0, 0)

        @pl.when(p > 0)
        def _():
            wait_write(1)
        start_gathers(s1, 1)
        wait_gathers(1)
        start_write(s1, 1)

        @pl.when(p + 1 < NSB // 2)
        def _():
            wait_write(0)
            start_gathers(s0 + 2, 0)
        return carry

    lax.fori_loop(0, NSB // 2, body, 0)
    wait_write(0)
    wait_write(1)


def kernel(tod, dow, emb, W, b):
    table, idx = _prep(emb, W, b, tod, dow)
    out = _sc_gather(idx.reshape(-1), table)
    return out.reshape(B, T, HIDDEN)
